# baseline (speedup 1.0000x reference)
import functools

import jax
import jax.numpy as jnp
from jax.experimental import pallas as pl
from jax.experimental.pallas import tpu as pltpu

_EPS = 1e-5
_VMEM = 48 * 1024 * 1024


def _cparams():
    return pltpu.CompilerParams(
        dimension_semantics=("parallel",),
        vmem_limit_bytes=_VMEM,
    )


def _conv_frame(a, w_ref, b_ref, c, wpad):
    pe = a.shape[1]
    xm = pltpu.roll(a, 1, 1)
    xp = pltpu.roll(a, pe - 1, 1)
    x3 = jnp.concatenate([xm, a, xp], axis=0)
    z = jnp.dot(w_ref[...], x3, preferred_element_type=jnp.float32)
    return (pltpu.roll(z[0:c], wpad, 1) + z[c:2 * c]
            + pltpu.roll(z[2 * c:3 * c], pe - wpad, 1) + b_ref[...])


def _stats(acc, mk, s_ref, q_ref):
    m = acc * mk
    s_ref[...] = jnp.sum(m, axis=1, keepdims=True)
    q_ref[...] = jnp.sum(m * m, axis=1, keepdims=True)


def _s1_kernel(x_ref, mk_ref, w_ref, b_ref, y_ref, s_ref, q_ref, *, c, wpad):
    acc = _conv_frame(x_ref[...], w_ref, b_ref, c, wpad)
    _stats(acc, mk_ref[...], s_ref, q_ref)
    y_ref[...] = acc


def _s2_kernel(y_ref, sc_ref, sh_ref, mk_ref, w_ref, b_ref,
               y2_ref, s_ref, q_ref, *, c, wpad):
    mk = mk_ref[...]
    a = jnp.maximum(y_ref[...] * sc_ref[...] + sh_ref[...], 0.0) * mk
    acc = _conv_frame(a, w_ref, b_ref, c, wpad)
    _stats(acc, mk, s_ref, q_ref)
    y2_ref[...] = acc


def _s3_kernel(y_ref, sc_ref, sh_ref, mk_ref, w_ref, b_ref, o_ref,
               *, c, wpad):
    mk = mk_ref[...]
    a = jnp.maximum(y_ref[...] * sc_ref[...] + sh_ref[...], 0.0) * mk
    acc = _conv_frame(a, w_ref, b_ref, c, wpad)
    o_ref[...] = jnp.maximum(acc + a, 0.0)


def _affine(s_parts, q_parts, count, gamma, beta):
    s = jnp.sum(s_parts, axis=0)[:, 0]
    q = jnp.sum(q_parts, axis=0)[:, 0]
    mean = s / count
    var = jnp.maximum(q / count - mean * mean, 0.0)
    scale = gamma / jnp.sqrt(var + _EPS)
    shift = beta - mean * scale
    return scale.reshape(-1, 1), shift.reshape(-1, 1)


def _frame_mask(pe, hpad, wpad):
    p = jnp.arange(pe, dtype=jnp.int32)[None, :]
    rp = p // wpad
    cp = p % wpad
    keep = ((p < hpad * wpad) & (rp >= 1) & (rp <= hpad - 2)
            & (cp >= 1) & (cp <= wpad - 2))
    return keep.astype(jnp.float32)


def kernel(x, w1, b1, w2, b2, w3, b3, g1, be1, g2, be2):
    x = x.astype(jnp.float32)
    n, c, h, w = x.shape
    hpad, wpad = h + 2, w + 2
    frame = hpad * wpad
    pe = -(-frame // 128) * 128
    if pe - frame < wpad + 1:
        pe += 128

    xf = jnp.pad(x, ((0, 0), (0, 0), (1, 1), (1, 1))).reshape(n, c, frame)
    xf = jnp.pad(xf, ((0, 0), (0, 0), (0, pe - frame)))

    wl1 = jnp.transpose(w1, (2, 0, 3, 1)).reshape(3 * c, 3 * c)
    wl2 = jnp.transpose(w2, (2, 0, 3, 1)).reshape(3 * c, 3 * c)
    wl3 = jnp.transpose(w3, (2, 0, 3, 1)).reshape(3 * c, 3 * c)
    bb1 = b1.reshape(c, 1)
    bb2 = b2.reshape(c, 1)
    bb3 = b3.reshape(c, 1)
    mask = _frame_mask(pe, hpad, wpad)

    act_spec = pl.BlockSpec((None, c, pe), lambda i: (i, 0, 0))
    w_spec = pl.BlockSpec((3 * c, 3 * c), lambda i: (0, 0))
    col_spec = pl.BlockSpec((c, 1), lambda i: (0, 0))
    mask_spec = pl.BlockSpec((1, pe), lambda i: (0, 0))
    stat_spec = pl.BlockSpec((None, c, 1), lambda i: (i, 0, 0))
    stat_shape = jax.ShapeDtypeStruct((n, c, 1), jnp.float32)

    y1, s1, q1 = pl.pallas_call(
        functools.partial(_s1_kernel, c=c, wpad=wpad),
        out_shape=(jax.ShapeDtypeStruct((n, c, pe), jnp.float32),
                   stat_shape, stat_shape),
        grid=(n,),
        in_specs=[act_spec, mask_spec, w_spec, col_spec],
        out_specs=(act_spec, stat_spec, stat_spec),
        compiler_params=_cparams(),
    )(xf, mask, wl1, bb1)

    sc1, sh1 = _affine(s1, q1, n * h * w, g1, be1)

    y2, s2, q2 = pl.pallas_call(
        functools.partial(_s2_kernel, c=c, wpad=wpad),
        out_shape=(jax.ShapeDtypeStruct((n, c, pe), jnp.float32),
                   stat_shape, stat_shape),
        grid=(n,),
        in_specs=[act_spec, col_spec, col_spec, mask_spec, w_spec, col_spec],
        out_specs=(act_spec, stat_spec, stat_spec),
        compiler_params=_cparams(),
    )(y1, sc1, sh1, mask, wl2, bb2)

    sc2, sh2 = _affine(s2, q2, n * h * w, g2, be2)

    out = pl.pallas_call(
        functools.partial(_s3_kernel, c=c, wpad=wpad),
        out_shape=jax.ShapeDtypeStruct((n, c, pe), jnp.float32),
        grid=(n,),
        in_specs=[act_spec, col_spec, col_spec, mask_spec, w_spec, col_spec],
        out_specs=act_spec,
        compiler_params=_cparams(),
    )(y2, sc2, sh2, mask, wl3, bb3)

    out = out[:, :, :frame].reshape(n, c, hpad, wpad)
    return out[:, :, 1:hpad - 1, 1:wpad - 1]


# speedup vs baseline: 1.2393x; 1.2393x over previous
import functools

import jax
import jax.numpy as jnp
from jax.experimental import pallas as pl
from jax.experimental.pallas import tpu as pltpu

_EPS = 1e-5
_VMEM = 48 * 1024 * 1024

_DT = jnp.bfloat16


def _cparams():
    return pltpu.CompilerParams(
        dimension_semantics=("parallel",),
        vmem_limit_bytes=_VMEM,
    )


def _conv_frame(a, w_ref, b_ref, c, wpad):
    pe = a.shape[1]
    xm = pltpu.roll(a, 1, 1)
    xp = pltpu.roll(a, pe - 1, 1)
    x3 = jnp.concatenate([xm, a, xp], axis=0)
    z = jnp.dot(w_ref[...], x3, preferred_element_type=jnp.float32)
    return (pltpu.roll(z[0:c], wpad, 1) + z[c:2 * c]
            + pltpu.roll(z[2 * c:3 * c], pe - wpad, 1) + b_ref[...])


def _stats(acc, mk, s_ref, q_ref):
    m = acc * mk
    s_ref[...] = jnp.sum(m, axis=1, keepdims=True)
    q_ref[...] = jnp.sum(m * m, axis=1, keepdims=True)


def _s1_kernel(x_ref, mk_ref, w_ref, b_ref, y_ref, s_ref, q_ref, xs_ref,
               *, c, h, w, wpad):
    xs_ref[...] = jnp.zeros(xs_ref.shape, xs_ref.dtype)
    for i in range(h):
        xs_ref[:, (i + 1) * wpad + 1:(i + 1) * wpad + 1 + w] = \
            x_ref[:, i * w:(i + 1) * w]
    acc = _conv_frame(xs_ref[...], w_ref, b_ref, c, wpad)
    _stats(acc, mk_ref[...], s_ref, q_ref)
    y_ref[...] = acc.astype(y_ref.dtype)


def _s2_kernel(y_ref, sc_ref, sh_ref, mk_ref, w_ref, b_ref,
               y2_ref, s_ref, q_ref, *, c, wpad):
    mk = mk_ref[...]
    yv = y_ref[...].astype(jnp.float32)
    a = jnp.maximum(yv * sc_ref[...] + sh_ref[...], 0.0) * mk
    acc = _conv_frame(a, w_ref, b_ref, c, wpad)
    _stats(acc, mk, s_ref, q_ref)
    y2_ref[...] = acc.astype(y2_ref.dtype)


def _s3_kernel(y_ref, sc_ref, sh_ref, mk_ref, w_ref, b_ref, o_ref,
               *, c, h, w, wpad):
    mk = mk_ref[...]
    yv = y_ref[...].astype(jnp.float32)
    a = jnp.maximum(yv * sc_ref[...] + sh_ref[...], 0.0) * mk
    acc = _conv_frame(a, w_ref, b_ref, c, wpad)
    res = jnp.maximum(acc + a, 0.0)
    for i in range(h):
        o_ref[:, i * w:(i + 1) * w] = \
            res[:, (i + 1) * wpad + 1:(i + 1) * wpad + 1 + w]


def _affine(s_parts, q_parts, count, gamma, beta):
    s = jnp.sum(s_parts, axis=0)[:, 0]
    q = jnp.sum(q_parts, axis=0)[:, 0]
    mean = s / count
    var = jnp.maximum(q / count - mean * mean, 0.0)
    scale = gamma / jnp.sqrt(var + _EPS)
    shift = beta - mean * scale
    return scale.reshape(-1, 1), shift.reshape(-1, 1)


def _frame_mask(pe, hpad, wpad):
    p = jnp.arange(pe, dtype=jnp.int32)[None, :]
    rp = p // wpad
    cp = p % wpad
    keep = ((p < hpad * wpad) & (rp >= 1) & (rp <= hpad - 2)
            & (cp >= 1) & (cp <= wpad - 2))
    return keep.astype(jnp.float32)


def kernel(x, w1, b1, w2, b2, w3, b3, g1, be1, g2, be2):
    x = x.astype(jnp.float32)
    n, c, h, w = x.shape
    hpad, wpad = h + 2, w + 2
    frame = hpad * wpad
    pe = -(-frame // 128) * 128
    if pe - frame < wpad + 1:
        pe += 128

    xf = x.reshape(n, c, h * w)

    wl1 = jnp.transpose(w1, (2, 0, 3, 1)).reshape(3 * c, 3 * c)
    wl2 = jnp.transpose(w2, (2, 0, 3, 1)).reshape(3 * c, 3 * c)
    wl3 = jnp.transpose(w3, (2, 0, 3, 1)).reshape(3 * c, 3 * c)
    bb1 = b1.reshape(c, 1)
    bb2 = b2.reshape(c, 1)
    bb3 = b3.reshape(c, 1)
    mask = _frame_mask(pe, hpad, wpad)

    act_spec = pl.BlockSpec((None, c, pe), lambda i: (i, 0, 0))
    dense_spec = pl.BlockSpec((None, c, h * w), lambda i: (i, 0, 0))
    w_spec = pl.BlockSpec((3 * c, 3 * c), lambda i: (0, 0))
    col_spec = pl.BlockSpec((c, 1), lambda i: (0, 0))
    mask_spec = pl.BlockSpec((1, pe), lambda i: (0, 0))
    stat_spec = pl.BlockSpec((None, c, 1), lambda i: (i, 0, 0))
    stat_shape = jax.ShapeDtypeStruct((n, c, 1), jnp.float32)

    y1, s1, q1 = pl.pallas_call(
        functools.partial(_s1_kernel, c=c, h=h, w=w, wpad=wpad),
        out_shape=(jax.ShapeDtypeStruct((n, c, pe), _DT),
                   stat_shape, stat_shape),
        grid=(n,),
        in_specs=[dense_spec, mask_spec, w_spec, col_spec],
        out_specs=(act_spec, stat_spec, stat_spec),
        scratch_shapes=[pltpu.VMEM((c, pe), jnp.float32)],
        compiler_params=_cparams(),
    )(xf, mask, wl1, bb1)

    sc1, sh1 = _affine(s1, q1, n * h * w, g1, be1)

    y2, s2, q2 = pl.pallas_call(
        functools.partial(_s2_kernel, c=c, wpad=wpad),
        out_shape=(jax.ShapeDtypeStruct((n, c, pe), _DT),
                   stat_shape, stat_shape),
        grid=(n,),
        in_specs=[act_spec, col_spec, col_spec, mask_spec, w_spec, col_spec],
        out_specs=(act_spec, stat_spec, stat_spec),
        compiler_params=_cparams(),
    )(y1, sc1, sh1, mask, wl2, bb2)

    sc2, sh2 = _affine(s2, q2, n * h * w, g2, be2)

    out = pl.pallas_call(
        functools.partial(_s3_kernel, c=c, h=h, w=w, wpad=wpad),
        out_shape=jax.ShapeDtypeStruct((n, c, h * w), jnp.float32),
        grid=(n,),
        in_specs=[act_spec, col_spec, col_spec, mask_spec, w_spec, col_spec],
        out_specs=dense_spec,
        compiler_params=_cparams(),
    )(y2, sc2, sh2, mask, wl3, bb3)

    return out.reshape(n, c, h, w)
